# baseline (speedup 1.0000x reference)
import functools

import jax
import jax.numpy as jnp
from jax import lax
from jax.experimental import pallas as pl
from jax.experimental.pallas import tpu as pltpu
from jax.experimental.pallas import tpu_sc as plsc

_NUM_TAGS = 100
_N_LAYERS = 12
_PREFIX = 50
_EMB = 768
_BATCH = 4096

_NC = 2
_NS = 16
_NW = _NC * _NS
_BW = _BATCH // _NW


def _sc_gather(table, tag_id):
  mesh = plsc.VectorSubcoreMesh(core_axis_name="c", subcore_axis_name="s")

  @functools.partial(
      pl.kernel,
      mesh=mesh,
      compiler_params=pltpu.CompilerParams(use_tc_tiling_on_sc=True),
      out_type=jax.ShapeDtypeStruct((_BATCH, _PREFIX, _EMB), jnp.float32),
      scratch_types=[
          pltpu.VMEM((_BW,), jnp.int32),
          pltpu.VMEM((1, _PREFIX, _EMB), jnp.float32),
          pltpu.VMEM((1, _PREFIX, _EMB), jnp.float32),
          pltpu.VMEM((1, _PREFIX, _EMB), jnp.float32),
          pltpu.SemaphoreType.DMA,
          pltpu.SemaphoreType.DMA,
          pltpu.SemaphoreType.DMA,
          pltpu.SemaphoreType.DMA,
          pltpu.SemaphoreType.DMA,
          pltpu.SemaphoreType.DMA,
      ],
  )
  def k(table_hbm, tag_hbm, out_hbm, tags_v, buf0, buf1, buf2,
        gsem0, gsem1, gsem2, ssem0, ssem1, ssem2):
    wid = lax.axis_index("s") * _NC + lax.axis_index("c")
    base = wid * _BW

    pltpu.sync_copy(tag_hbm.at[pl.ds(base, _BW)], tags_v)

    bufs = (buf0, buf1, buf2)
    gsems = (gsem0, gsem1, gsem2)
    ssems = (ssem0, ssem1, ssem2)

    def start_gather(blk, b):
      pltpu.async_copy(table_hbm.at[pl.ds(blk, 1)], bufs[b], gsems[b])

    def wait_gather(b):
      pltpu.make_async_copy(table_hbm.at[pl.ds(0, 1)], bufs[b],
                            gsems[b]).wait()

    def start_scatter(item, b):
      pltpu.async_copy(bufs[b], out_hbm.at[pl.ds(base + item, 1)], ssems[b])

    def wait_scatter(b):
      pltpu.make_async_copy(bufs[b], out_hbm.at[pl.ds(0, 1)], ssems[b]).wait()

    def item_step(item, vtag, b, first_round):
      if first_round:
        pass
      else:
        wait_scatter(b)
      start_gather(vtag, b)
      wait_gather(b)
      start_scatter(item, b)

    def group(g, carry):
      base_i = g * 48
      for half in range(3):
        v = tags_v[pl.ds(base_i + half * 16, 16)] * _N_LAYERS
        for i in range(16):
          ii = half * 16 + i
          b = ii % 3
          if ii < 3:
            @pl.when(g > 0)
            def _():
              wait_scatter(b)
            start_gather(v[i], b)
            wait_gather(b)
            start_scatter(base_i + ii, b)
          else:
            item_step(base_i + ii, v[i], b, False)
      return carry

    lax.fori_loop(0, 2, group, None)
    for half in range(2):
      v = tags_v[pl.ds(96 + half * 16, 16)] * _N_LAYERS
      for i in range(16):
        ii = half * 16 + i
        item_step(96 + ii, v[i], ii % 3, False)
    wait_scatter(0)
    wait_scatter(1)
    wait_scatter(2)

  return k(table, tag_id)


def kernel(prefixes, tag_id):
  table = prefixes.reshape((_NUM_TAGS + 1) * _N_LAYERS, _PREFIX, _EMB)
  return _sc_gather(table, tag_id)


# speedup vs baseline: 2.2619x; 2.2619x over previous
import functools

import jax
import jax.numpy as jnp
from jax import lax
from jax.experimental import pallas as pl
from jax.experimental.pallas import tpu as pltpu
from jax.experimental.pallas import tpu_sc as plsc

_NUM_TAGS = 100
_N_LAYERS = 12
_PREFIX = 50
_EMB = 768
_BATCH = 4096

_NC = 2
_NS = 16
_NW = _NC * _NS
_BW = _BATCH // _NW


def _sc_gather(table, tag_id):
  mesh = plsc.VectorSubcoreMesh(core_axis_name="c", subcore_axis_name="s")

  @functools.partial(
      pl.kernel,
      mesh=mesh,
      compiler_params=pltpu.CompilerParams(use_tc_tiling_on_sc=True),
      out_type=jax.ShapeDtypeStruct((_PREFIX, _BATCH, _EMB), jnp.float32),
      scratch_types=[
          pltpu.VMEM((_BW,), jnp.int32),
          pltpu.VMEM((_PREFIX, _EMB), jnp.float32),
          pltpu.VMEM((_PREFIX, _EMB), jnp.float32),
          pltpu.VMEM((_PREFIX, _EMB), jnp.float32),
          pltpu.SemaphoreType.DMA,
          pltpu.SemaphoreType.DMA,
          pltpu.SemaphoreType.DMA,
          pltpu.SemaphoreType.DMA,
          pltpu.SemaphoreType.DMA,
          pltpu.SemaphoreType.DMA,
      ],
  )
  def k(table_hbm, tag_hbm, out_hbm, tags_v, buf0, buf1, buf2,
        gsem0, gsem1, gsem2, ssem0, ssem1, ssem2):
    wid = lax.axis_index("s") * _NC + lax.axis_index("c")
    base = wid * _BW

    pltpu.sync_copy(tag_hbm.at[pl.ds(base, _BW)], tags_v)

    bufs = (buf0, buf1, buf2)
    gsems = (gsem0, gsem1, gsem2)
    ssems = (ssem0, ssem1, ssem2)

    def start_gather(blk, b):
      pltpu.async_copy(table_hbm.at[blk], bufs[b], gsems[b])

    def wait_gather(b):
      pltpu.make_async_copy(table_hbm.at[0], bufs[b], gsems[b]).wait()

    def start_scatter(item, b):
      pltpu.async_copy(bufs[b], out_hbm.at[:, base + item, :], ssems[b])

    def wait_scatter(b):
      pltpu.make_async_copy(bufs[b], out_hbm.at[:, 0, :], ssems[b]).wait()

    def drain_for(local_ii, g, b):
      if local_ii < 3:
        @pl.when(g > 0)
        def _():
          wait_scatter(b)
      else:
        wait_scatter(b)

    def block16(v, base_i, c, g):
      for i in range(16):
        ii = c * 16 + i
        item = base_i + ii
        b = ii % 3
        if i == 0:
          drain_for(ii, g, b)
          start_gather(v[0], b)
        if i < 15:
          bn = (ii + 1) % 3
          drain_for(ii + 1, g, bn)
          start_gather(v[i + 1], bn)
        wait_gather(b)
        start_scatter(item, b)

    def group(g, carry):
      base_i = g * 48
      for c in range(3):
        v = tags_v[pl.ds(base_i + c * 16, 16)]
        block16(v, base_i, c, g)
      return carry

    lax.fori_loop(0, 2, group, None)
    for c in range(2):
      v = tags_v[pl.ds(96 + c * 16, 16)]
      block16(v, 96, c, 1)
    wait_scatter(0)
    wait_scatter(1)
    wait_scatter(2)

  return k(table, tag_id)


def kernel(prefixes, tag_id):
  table = prefixes[:, 0]
  out3 = _sc_gather(table, tag_id)
  return out3.transpose(1, 0, 2)
